# baseline (speedup 1.0000x reference)
import functools

import jax
import jax.numpy as jnp
from jax import lax
from jax.experimental import pallas as pl
from jax.experimental.pallas import tpu as pltpu
from jax.experimental.pallas import tpu_sc as plsc

_NUM_EMB = 1000000
_DIM = 32
_BATCH = 16384
_QROWS = _NUM_EMB * _DIM // 128

_NC = 2
_NS = 16
_NW = _NC * _NS
_BPW = _BATCH // _NW

_sc_mesh = plsc.VectorSubcoreMesh(core_axis_name="c", subcore_axis_name="s")


@functools.partial(
    pl.kernel,
    mesh=_sc_mesh,
    compiler_params=pltpu.CompilerParams(needs_layout_passes=False),
    out_type=[jax.ShapeDtypeStruct((_DIM, _BATCH), jnp.float32)] * 3,
    scratch_types=[
        pltpu.VMEM((_BPW,), jnp.int32),
        pltpu.VMEM((_BPW,), jnp.int32),
        pltpu.VMEM((_BPW, 128), jnp.float32),
        pltpu.VMEM((_DIM, _BPW), jnp.float32),
        pltpu.SemaphoreType.DMA,
    ],
)
def _gather3(idx_hbm, tw_hbm, pw_hbm, vw_hbm, out_t, out_p, out_v,
             idx_v, qrow_v, rows_v, stage_v, sem):
    wid = lax.axis_index("s") * _NC + lax.axis_index("c")
    base = wid * _BPW
    pltpu.sync_copy(idx_hbm.at[pl.ds(base, _BPW)], idx_v)
    lanes = lax.iota(jnp.int32, 16)

    def qrow_body(c, _):
        v = idx_v[pl.ds(c * 16, 16)]
        qrow_v[pl.ds(c * 16, 16)] = v >> 2
        return ()

    lax.fori_loop(0, _BPW // 16, qrow_body, (), unroll=False)

    for src, out in ((tw_hbm, out_t), (pw_hbm, out_p), (vw_hbm, out_v)):
        pltpu.async_copy(src.at[qrow_v], rows_v, sem).wait()

        def extract_body(c, _):
            v = idx_v[pl.ds(c * 16, 16)]
            sub = (v & 3) * 32
            rvec = lanes + c * 16
            for d in range(_DIM):
                vals = plsc.load_gather(rows_v, [rvec, sub + d])
                stage_v[d, pl.ds(c * 16, 16)] = vals
            return ()

        lax.fori_loop(0, _BPW // 16, extract_body, (), unroll=False)
        pltpu.sync_copy(stage_v, out.at[:, pl.ds(base, _BPW)])


def _trig_body(t_ref, p_ref, v_ref, ha_ref, hai_ref, hb_ref, hbi_ref):
    t = t_ref[...]
    p = p_ref[...]
    v = v_ref[...]
    st = jnp.sin(t)
    stsp = st * jnp.sin(p)
    ha_ref[...] = jnp.cos(t)
    hai_ref[...] = st * jnp.cos(p)
    hb_ref[...] = stsp * jnp.cos(v)
    hbi_ref[...] = stsp * jnp.sin(v)


_TBLK = 2048


def _trig(theta, phi, varphi):
    spec = pl.BlockSpec((_DIM, _TBLK), lambda i: (0, i))
    out = jax.ShapeDtypeStruct((_DIM, _BATCH), jnp.float32)
    return pl.pallas_call(
        _trig_body,
        grid=(_BATCH // _TBLK,),
        in_specs=[spec, spec, spec],
        out_specs=[spec, spec, spec, spec],
        out_shape=[out, out, out, out],
    )(theta, phi, varphi)


@jax.jit
def kernel(h_idx, theta_w, phi_w, varphi_w):
    idx = h_idx.astype(jnp.int32)
    tw = theta_w.reshape(_QROWS, 128)
    pw = phi_w.reshape(_QROWS, 128)
    vw = varphi_w.reshape(_QROWS, 128)
    theta, phi, varphi = _gather3(idx, tw, pw, vw)
    ha, hai, hb, hbi = _trig(theta, phi, varphi)

    def fin(x):
        return x.T

    return ((fin(ha), fin(hai)), (fin(hb), fin(hbi)))
